# baseline (speedup 1.0000x reference)
import functools

import jax
import jax.numpy as jnp
from jax.experimental import pallas as pl
from jax.experimental.pallas import tpu as pltpu

NSEG = 256
BLK = 4096
WIN = 64


def _body(n_rows, meta_ref, x_ref, b_ref, w1_ref, b1_ref, w2_ref, c_ref,
          out_ref, accn, accd):
    i = pl.program_id(0)
    nblk = pl.num_programs(0)

    @pl.when(i == 0)
    def _init():
        accn[...] = jnp.zeros_like(accn)
        accd[...] = jnp.zeros_like(accd)

    xb = x_ref[...].astype(jnp.bfloat16)
    h = jnp.tanh(
        jnp.dot(xb, w1_ref[...].astype(jnp.bfloat16),
                preferred_element_type=jnp.float32)
        + b1_ref[...]
    ).astype(jnp.bfloat16)
    s_row = jax.lax.dot_general(
        w2_ref[...].astype(jnp.bfloat16), h, (((1,), (1,)), ((), ())),
        preferred_element_type=jnp.float32)
    exb_row = jnp.exp(s_row - c_ref[0, 0]).astype(jnp.bfloat16)

    if n_rows % BLK:
        tail = n_rows - (n_rows // BLK) * BLK

        def _mask(args):
            xb_, ex_ = args
            col = jax.lax.broadcasted_iota(jnp.int32, (1, BLK), 1)
            ex_ = jnp.where(col < tail, ex_, jnp.bfloat16(0.0))
            row = jax.lax.broadcasted_iota(jnp.int32, (BLK, 1), 0)
            xb_ = jnp.where(row < tail, xb_, jnp.bfloat16(0.0))
            return xb_, ex_

        xb, exb_row = jax.lax.cond(
            i == nblk - 1, _mask, lambda a: a, (xb, exb_row))

    b_row = b_ref[0].astype(jnp.int16)
    base = pl.multiple_of(meta_ref[2 * i], 8)
    ok = meta_ref[2 * i + 1]
    ones_rhs = jnp.ones((BLK, 128), jnp.bfloat16)

    @pl.when(ok == 1)
    def _windowed():
        rel = b_row - base.astype(jnp.int16)
        ohw = jnp.where(
            jax.lax.broadcasted_iota(jnp.int16, (WIN, BLK), 0) == rel,
            jnp.broadcast_to(exb_row, (WIN, BLK)), jnp.bfloat16(0.0))
        accn[pl.ds(base, WIN), :] += jnp.dot(
            ohw, xb, preferred_element_type=jnp.float32)
        accd[pl.ds(base, WIN), :] += jnp.dot(
            ohw, ones_rhs, preferred_element_type=jnp.float32)

    @pl.when(ok == 0)
    def _full():
        ohw = jnp.where(
            jax.lax.broadcasted_iota(jnp.int16, (NSEG, BLK), 0) == b_row,
            jnp.broadcast_to(exb_row, (NSEG, BLK)), jnp.bfloat16(0.0))
        accn[...] += jnp.dot(ohw, xb, preferred_element_type=jnp.float32)
        accd[...] += jnp.dot(ohw, ones_rhs, preferred_element_type=jnp.float32)

    @pl.when(i == nblk - 1)
    def _fin():
        out_ref[...] = accn[...] / (accd[...] + 1e-16)


@functools.partial(jax.jit, static_argnames=())
def kernel(x, batch, W1, b1, W2, b2):
    n = x.shape[0]
    nblk = (n + BLK - 1) // BLK
    npad = nblk * BLK
    b32 = batch.astype(jnp.int32)
    bp = jnp.concatenate(
        [b32, jnp.broadcast_to(b32[-1:], (npad - n,))])
    bp3 = bp.reshape(nblk, 1, BLK)
    first = bp3[:, 0, 0]
    last = bp3[:, 0, -1]
    base = jnp.minimum((first // 8) * 8, NSEG - WIN)
    ok = (last < base + WIN).astype(jnp.int32)
    meta = jnp.stack([base, ok], axis=1).reshape(-1)
    w2row = W2.reshape(1, -1)
    c = jnp.sum(jnp.abs(w2row)).reshape(1, 1)
    b1r = b1.reshape(1, -1)

    grid_spec = pltpu.PrefetchScalarGridSpec(
        num_scalar_prefetch=1,
        grid=(nblk,),
        in_specs=[
            pl.BlockSpec((BLK, 128), lambda i, m: (i, 0)),
            pl.BlockSpec((1, 1, BLK), lambda i, m: (i, 0, 0)),
            pl.BlockSpec((128, 64), lambda i, m: (0, 0)),
            pl.BlockSpec((1, 64), lambda i, m: (0, 0)),
            pl.BlockSpec((1, 64), lambda i, m: (0, 0)),
            pl.BlockSpec((1, 1), lambda i, m: (0, 0)),
        ],
        out_specs=pl.BlockSpec((NSEG, 128), lambda i, m: (0, 0)),
        scratch_shapes=[
            pltpu.VMEM((NSEG, 128), jnp.float32),
            pltpu.VMEM((NSEG, 128), jnp.float32),
        ],
    )

    return pl.pallas_call(
        functools.partial(_body, n),
        grid_spec=grid_spec,
        out_shape=jax.ShapeDtypeStruct((NSEG, 128), jnp.float32),
    )(meta, x, bp3, W1, b1r, w2row, c)


# speedup vs baseline: 1.1188x; 1.1188x over previous
import functools

import jax
import jax.numpy as jnp
from jax.experimental import pallas as pl
from jax.experimental.pallas import tpu as pltpu

NSEG = 256
BLK = 8192
WIN = 64


def _body(n_rows, meta_ref, x_ref, b_ref, w1_ref, b1_ref, w2_ref,
          out_ref, accn, accd):
    i = pl.program_id(0)
    nblk = pl.num_programs(0)

    @pl.when(i == 0)
    def _init():
        accn[...] = jnp.zeros_like(accn)
        accd[...] = jnp.zeros_like(accd)

    xb = x_ref[...].astype(jnp.bfloat16)
    h = jnp.tanh(
        jnp.dot(xb, w1_ref[...].astype(jnp.bfloat16),
                preferred_element_type=jnp.float32)
        + b1_ref[...]
    ).astype(jnp.bfloat16)
    w2 = w2_ref[...]
    c = jnp.sum(jnp.abs(w2))
    s_row = jax.lax.dot_general(
        w2.astype(jnp.bfloat16), h, (((1,), (1,)), ((), ())),
        preferred_element_type=jnp.float32)
    exb_row = jnp.exp(s_row - c).astype(jnp.bfloat16)

    if n_rows % BLK:
        tail = n_rows - (n_rows // BLK) * BLK

        def _mask(args):
            xb_, ex_ = args
            col = jax.lax.broadcasted_iota(jnp.int32, (1, BLK), 1)
            ex_ = jnp.where(col < tail, ex_, jnp.bfloat16(0.0))
            row = jax.lax.broadcasted_iota(jnp.int32, (BLK, 1), 0)
            xb_ = jnp.where(row < tail, xb_, jnp.bfloat16(0.0))
            return xb_, ex_

        xb, exb_row = jax.lax.cond(
            i == nblk - 1, _mask, lambda a: a, (xb, exb_row))

    b_row = b_ref[...].astype(jnp.int16)
    base = pl.multiple_of(meta_ref[2 * i], 8)
    ok = meta_ref[2 * i + 1]
    ones_rhs = jnp.ones((BLK, 128), jnp.bfloat16)

    @pl.when(ok == 1)
    def _windowed():
        rel = b_row - base.astype(jnp.int16)
        ohw = jnp.where(
            jax.lax.broadcasted_iota(jnp.int16, (WIN, BLK), 0) == rel,
            jnp.broadcast_to(exb_row, (WIN, BLK)), jnp.bfloat16(0.0))
        accn[pl.ds(base, WIN), :] += jnp.dot(
            ohw, xb, preferred_element_type=jnp.float32)
        accd[pl.ds(base, WIN), :] += jnp.dot(
            ohw, ones_rhs, preferred_element_type=jnp.float32)

    @pl.when(ok == 0)
    def _full():
        ohw = jnp.where(
            jax.lax.broadcasted_iota(jnp.int16, (NSEG, BLK), 0) == b_row,
            jnp.broadcast_to(exb_row, (NSEG, BLK)), jnp.bfloat16(0.0))
        accn[...] += jnp.dot(ohw, xb, preferred_element_type=jnp.float32)
        accd[...] += jnp.dot(ohw, ones_rhs, preferred_element_type=jnp.float32)

    @pl.when(i == nblk - 1)
    def _fin():
        out_ref[...] = accn[...] / (accd[...] + 1e-16)


@functools.partial(jax.jit, static_argnames=())
def kernel(x, batch, W1, b1, W2, b2):
    n = x.shape[0]
    nblk = (n + BLK - 1) // BLK
    b32 = batch.astype(jnp.int32)
    firsts = b32[jnp.arange(nblk) * BLK]
    lasts = b32[jnp.minimum(jnp.arange(nblk) * BLK + BLK - 1, n - 1)]
    base = jnp.minimum((firsts // 8) * 8, NSEG - WIN)
    ok = (lasts < base + WIN).astype(jnp.int32)
    meta = jnp.stack([base, ok], axis=1).reshape(-1)

    grid_spec = pltpu.PrefetchScalarGridSpec(
        num_scalar_prefetch=1,
        grid=(nblk,),
        in_specs=[
            pl.BlockSpec((BLK, 128), lambda i, m: (i, 0)),
            pl.BlockSpec((1, BLK), lambda i, m: (0, i)),
            pl.BlockSpec((128, 64), lambda i, m: (0, 0)),
            pl.BlockSpec((1, 64), lambda i, m: (0, 0)),
            pl.BlockSpec((1, 64), lambda i, m: (0, 0)),
        ],
        out_specs=pl.BlockSpec((NSEG, 128), lambda i, m: (0, 0)),
        scratch_shapes=[
            pltpu.VMEM((NSEG, 128), jnp.float32),
            pltpu.VMEM((NSEG, 128), jnp.float32),
        ],
    )

    return pl.pallas_call(
        functools.partial(_body, n),
        grid_spec=grid_spec,
        out_shape=jax.ShapeDtypeStruct((NSEG, 128), jnp.float32),
    )(meta, x, b32.reshape(1, n), W1, b1.reshape(1, -1), W2.reshape(1, -1))
